# baseline (speedup 1.0000x reference)
import functools

import jax
import jax.numpy as jnp
from jax import lax
from jax.experimental import pallas as pl
from jax.experimental.pallas import tpu as pltpu
from jax.experimental.pallas import tpu_sc as plsc

D_MODEL = 64
NC = 2
NS = 16
NW = NC * NS
CHUNK = 256
NB = 3


def _sc_gather(n_flat: int):
    b_per_w = n_flat // NW
    n_chunks = b_per_w // CHUNK
    mesh = plsc.VectorSubcoreMesh(core_axis_name="c", subcore_axis_name="s")

    @functools.partial(
        pl.kernel,
        out_type=jax.ShapeDtypeStruct((n_flat // 16, 16, D_MODEL), jnp.float32),
        mesh=mesh,
        scratch_types=[
            pltpu.VMEM((b_per_w,), jnp.int32),
            [pltpu.VMEM((CHUNK // 16, 16, D_MODEL), jnp.float32)
             for _ in range(NB)],
            [pltpu.SemaphoreType.DMA for _ in range(NB)],
            [pltpu.SemaphoreType.DMA for _ in range(NB)],
        ],
        compiler_params=pltpu.CompilerParams(use_tc_tiling_on_sc=True),
    )
    def body(table_hbm, idx_hbm, out_hbm, idx_all, rows, sg, so):
        wid = lax.axis_index("s") * NC + lax.axis_index("c")
        base = wid * b_per_w
        pltpu.sync_copy(idx_hbm.at[pl.ds(base, b_per_w)], idx_all)

        def gather(c, b):
            def grp16(g, _):
                vec = idx_all[pl.ds(c * CHUNK + g * 16, 16)]
                for lane in range(16):
                    i = vec[lane]
                    pltpu.async_copy(
                        table_hbm.at[i // 16, pl.ds(i % 16, 1), :],
                        rows[b].at[g, pl.ds(lane, 1), :],
                        sg[b],
                    )
                return ()

            lax.fori_loop(0, CHUNK // 16, grp16, ())

        def wait_gather(b):
            pltpu.make_async_copy(
                out_hbm.at[pl.ds(0, CHUNK // 16)], rows[b], sg[b]
            ).wait()

        def put(c, b):
            off = pl.multiple_of((base + c * CHUNK) // 16, 8)
            pltpu.async_copy(
                rows[b], out_hbm.at[pl.ds(off, CHUNK // 16)], so[b]
            )

        def wait_put(b):
            pltpu.make_async_copy(
                rows[b], out_hbm.at[pl.ds(0, CHUNK // 16)], so[b]
            ).wait()

        gather(0, 0)
        gather(1, 1)
        wait_gather(0)
        put(0, 0)
        gather(2, 2)
        wait_gather(1)
        put(1, 1)
        wait_put(0)
        gather(3, 0)
        wait_gather(2)
        put(2, 2)

        def pipe3(g, _):
            for k in range(3):
                c = g * 3 + 4 + k
                b = (1 + k) % 3
                wait_put(b)
                gather(c, b)
                wait_gather(k % 3)
                put(c - 1, k % 3)
            return ()

        lax.fori_loop(0, (n_chunks - 4) // 3, pipe3, ())

        c_last = n_chunks - 1
        wait_gather(c_last % 3)
        put(c_last, c_last % 3)
        for b in range(NB):
            wait_put(b)

    return body


def kernel(x, table):
    b, l = x.shape
    v, d = table.shape
    flat = x.reshape(-1).astype(jnp.int32)
    out = _sc_gather(b * l)(table.reshape(v // 16, 16, d), flat)
    return out.reshape(b, l, D_MODEL)


# speedup vs baseline: 1.0027x; 1.0027x over previous
import functools

import jax
import jax.numpy as jnp
from jax import lax
from jax.experimental import pallas as pl
from jax.experimental.pallas import tpu as pltpu
from jax.experimental.pallas import tpu_sc as plsc

D_MODEL = 64
NC = 2
NS = 16
NW = NC * NS
CHUNK = 256
NB = 3


def _sc_gather(n_flat: int):
    b_per_w = n_flat // NW
    n_chunks = b_per_w // CHUNK
    mesh = plsc.VectorSubcoreMesh(core_axis_name="c", subcore_axis_name="s")

    @functools.partial(
        pl.kernel,
        out_type=jax.ShapeDtypeStruct((n_flat, D_MODEL), jnp.float32),
        mesh=mesh,
        scratch_types=[
            pltpu.VMEM((b_per_w,), jnp.int32),
            [pltpu.VMEM((CHUNK, D_MODEL), jnp.float32) for _ in range(NB)],
            [pltpu.SemaphoreType.DMA for _ in range(NB)],
            [pltpu.SemaphoreType.DMA for _ in range(NB)],
        ],
        compiler_params=pltpu.CompilerParams(use_tc_tiling_on_sc=True),
    )
    def body(table_hbm, idx_hbm, out_hbm, idx_all, rows, sg, so):
        wid = lax.axis_index("s") * NC + lax.axis_index("c")
        base = wid * b_per_w
        pltpu.sync_copy(idx_hbm.at[pl.ds(base, b_per_w)], idx_all)

        def gather(c, b):
            def grp16(g, _):
                vec = idx_all[pl.ds(c * CHUNK + g * 16, 16)]
                for lane in range(16):
                    i = vec[lane]
                    pltpu.async_copy(
                        table_hbm.at[i // 16, pl.ds(i % 16, 1), :],
                        rows[b].at[pl.ds(g * 16 + lane, 1), :],
                        sg[b],
                    )
                return ()

            lax.fori_loop(0, CHUNK // 16, grp16, ())

        def wait_gather(b):
            pltpu.make_async_copy(
                out_hbm.at[pl.ds(0, CHUNK)], rows[b], sg[b]
            ).wait()

        def put(c, b):
            off = pl.multiple_of(base + c * CHUNK, 8)
            pltpu.async_copy(rows[b], out_hbm.at[pl.ds(off, CHUNK)], so[b])

        def wait_put(b):
            off = pl.multiple_of(base, 8)
            pltpu.make_async_copy(
                rows[b], out_hbm.at[pl.ds(off, CHUNK)], so[b]
            ).wait()

        gather(0, 0)
        gather(1, 1)
        wait_gather(0)
        put(0, 0)
        gather(2, 2)
        wait_gather(1)
        put(1, 1)
        wait_put(0)
        gather(3, 0)
        wait_gather(2)
        put(2, 2)

        def pipe3(g, _):
            for k in range(3):
                c = g * 3 + 4 + k
                b = (1 + k) % 3
                wait_put(b)
                gather(c, b)
                wait_gather(k % 3)
                put(c - 1, k % 3)
            return ()

        lax.fori_loop(0, (n_chunks - 4) // 3, pipe3, ())

        c_last = n_chunks - 1
        wait_gather(c_last % 3)
        put(c_last, c_last % 3)
        for b in range(NB):
            wait_put(b)

    return body


def kernel(x, table):
    b, l = x.shape
    v, d = table.shape
    flat = x.reshape(-1).astype(jnp.int32)
    out = _sc_gather(b * l)(table.reshape(v // 16, 16, d), flat)
    return out.reshape(b, l, D_MODEL)
